# baseline (speedup 1.0000x reference)
import functools

import jax
import jax.numpy as jnp
from jax import lax
from jax.experimental import pallas as pl
from jax.experimental.pallas import tpu as pltpu
from jax.experimental.pallas import tpu_sc as plsc

K_ROWS = 100000
NZ = 64
PADW = 128

_BK = 10000
_NB = K_ROWS // _BK


def _sigma_body(u_ref, w_ref, o_ref, t_acc, g_acc):
    i = pl.program_id(0)

    @pl.when(i == 0)
    def _init():
        t_acc[...] = jnp.zeros_like(t_acc)
        g_acc[...] = jnp.zeros_like(g_acc)

    wb = w_ref[...]
    ub = u_ref[...]
    t_acc[...] += jnp.sum(wb * ub, axis=0, keepdims=True)
    g_acc[...] += lax.dot_general(
        wb, wb, (((0,), (0,)), ((), ())),
        preferred_element_type=jnp.float32,
        precision=lax.Precision.HIGHEST)

    @pl.when(i == _NB - 1)
    def _fin():
        t = t_acc[...]
        g = g_acc[...]
        v = t / (jnp.sqrt(jnp.sum(t * t)) + 1e-12)
        gv = lax.dot_general(
            v, g, (((1,), (0,)), ((), ())),
            preferred_element_type=jnp.float32,
            precision=lax.Precision.HIGHEST)
        sig2 = jnp.sum(gv * v[0, :])
        o_ref[0, 0] = 1.0 / jnp.sqrt(sig2)


def _recip_sigma(weight, u):
    return pl.pallas_call(
        _sigma_body,
        grid=(_NB,),
        in_specs=[
            pl.BlockSpec((_BK, 1), lambda i: (i, 0)),
            pl.BlockSpec((_BK, NZ), lambda i: (i, 0)),
        ],
        out_specs=pl.BlockSpec(memory_space=pltpu.SMEM),
        out_shape=jax.ShapeDtypeStruct((1, 1), jnp.float32),
        scratch_shapes=[
            pltpu.VMEM((1, NZ), jnp.float32),
            pltpu.VMEM((NZ, NZ), jnp.float32),
        ],
        compiler_params=pltpu.CompilerParams(
            dimension_semantics=("arbitrary",)),
    )(u.reshape(K_ROWS, 1), weight)


_NQ = 4096
_SEQ = 50
_BG = 128
_BKF = _BG * _SEQ
_NBF = _NQ // _BG


def _final_body(r_ref, par_ref, g_ref, o_ref):
    r = r_ref[0, 0]
    g = g_ref[...]
    par = par_ref[...]
    p = par.reshape(_BKF, 1)
    sel = (g[:, :NZ] + p * (g[:, NZ:] - g[:, :NZ])) * r
    o_ref[...] = sel


def _select_scale(gathered, parity, recip):
    return pl.pallas_call(
        _final_body,
        grid=(_NBF,),
        in_specs=[
            pl.BlockSpec(memory_space=pltpu.SMEM),
            pl.BlockSpec((1, 1, _BKF), lambda i: (i, 0, 0)),
            pl.BlockSpec((_BKF, PADW), lambda i: (i, 0)),
        ],
        out_specs=pl.BlockSpec((_BKF, NZ), lambda i: (i, 0)),
        out_shape=jax.ShapeDtypeStruct((_B_TOTAL, NZ), jnp.float32),
    )(recip, parity, gathered)


_NC = 2
_NS = 16
_NW = _NC * _NS
_B_TOTAL = 4096 * 50
_PER_W = _B_TOTAL // _NW
_RCH = 128
_NG = 5
_R = _RCH * _NG
_CH = _PER_W // _R
_IROWS_W = _PER_W // _RCH


def _gather_body(wpad_hbm, idx_hbm, out_hbm, idx_v, rows_v, sem):
    c = lax.axis_index("c")
    s = lax.axis_index("s")
    wid = s * _NC + c
    rbase = wid * _PER_W

    pltpu.sync_copy(idx_hbm.at[wid], idx_v)

    def chunk(i, carry):
        roff = rbase + i * _R
        cps = [
            pltpu.async_copy(
                wpad_hbm.at[idx_v.at[i * _NG + j]],
                rows_v.at[pl.ds(j * _RCH, _RCH)],
                sem)
            for j in range(_NG)
        ]
        for cp in cps:
            cp.wait()
        pltpu.sync_copy(rows_v, out_hbm.at[pl.ds(roff, _R)])
        return carry

    lax.fori_loop(0, _CH, chunk, 0)


def _gather(wpad, idx2d):
    mesh = plsc.VectorSubcoreMesh(
        core_axis_name="c", subcore_axis_name="s",
        num_cores=_NC, num_subcores=_NS)
    fn = pl.kernel(
        _gather_body,
        out_type=jax.ShapeDtypeStruct((_B_TOTAL, PADW), jnp.float32),
        mesh=mesh,
        scratch_types=[
            pltpu.VMEM((_IROWS_W, _RCH), jnp.int32),
            pltpu.VMEM((_R, PADW), jnp.float32),
            pltpu.SemaphoreType.DMA,
        ],
    )
    return fn(wpad, idx2d)


def kernel(weight, u, k):
    ki = k.astype(jnp.int32)
    wpair = weight.reshape(K_ROWS // 2, PADW)
    idx2d = (ki >> 1).reshape(_NW, _IROWS_W, _RCH)
    g = _gather(wpair, idx2d)
    recip = _recip_sigma(weight, u)
    parity = (ki & 1).astype(jnp.float32).reshape(_NBF, 1, _BKF)
    out2d = _select_scale(g, parity, recip)
    return out2d.reshape(_NQ, _SEQ, NZ)


# speedup vs baseline: 1.1646x; 1.1646x over previous
import functools

import jax
import jax.numpy as jnp
from jax import lax
from jax.experimental import pallas as pl
from jax.experimental.pallas import tpu as pltpu
from jax.experimental.pallas import tpu_sc as plsc

K_ROWS = 100000
NZ = 64
PADW = 128

_BK = 10000
_NB = K_ROWS // _BK


def _sigma_body(u_ref, w_ref, o_ref, t_acc, g_acc):
    i = pl.program_id(0)

    @pl.when(i == 0)
    def _init():
        t_acc[...] = jnp.zeros_like(t_acc)
        g_acc[...] = jnp.zeros_like(g_acc)

    wb = w_ref[...]
    ub = u_ref[...]
    t_acc[...] += jnp.sum(wb * ub, axis=0, keepdims=True)
    g_acc[...] += lax.dot_general(
        wb, wb, (((0,), (0,)), ((), ())),
        preferred_element_type=jnp.float32,
        precision=lax.Precision.HIGHEST)

    @pl.when(i == _NB - 1)
    def _fin():
        t = t_acc[...]
        g = g_acc[...]
        v = t / (jnp.sqrt(jnp.sum(t * t)) + 1e-12)
        gv = lax.dot_general(
            v, g, (((1,), (0,)), ((), ())),
            preferred_element_type=jnp.float32,
            precision=lax.Precision.HIGHEST)
        sig2 = jnp.sum(gv * v[0, :])
        o_ref[0, 0] = 1.0 / jnp.sqrt(sig2)


def _recip_sigma(weight, u):
    return pl.pallas_call(
        _sigma_body,
        grid=(_NB,),
        in_specs=[
            pl.BlockSpec((_BK, 1), lambda i: (i, 0)),
            pl.BlockSpec((_BK, NZ), lambda i: (i, 0)),
        ],
        out_specs=pl.BlockSpec(memory_space=pltpu.SMEM),
        out_shape=jax.ShapeDtypeStruct((1, 1), jnp.float32),
        scratch_shapes=[
            pltpu.VMEM((1, NZ), jnp.float32),
            pltpu.VMEM((NZ, NZ), jnp.float32),
        ],
        compiler_params=pltpu.CompilerParams(
            dimension_semantics=("arbitrary",)),
    )(u.reshape(K_ROWS, 1), weight)


_NQ = 4096
_SEQ = 50


def _pad_body(r_ref, w_ref, o_ref):
    r = r_ref[0, 0]
    blk = w_ref[...] * r
    o_ref[...] = jnp.concatenate([blk, jnp.zeros_like(blk)], axis=1)


def _pad_scale(weight, recip):
    return pl.pallas_call(
        _pad_body,
        grid=(_NB,),
        in_specs=[
            pl.BlockSpec(memory_space=pltpu.SMEM),
            pl.BlockSpec((_BK, NZ), lambda i: (i, 0)),
        ],
        out_specs=pl.BlockSpec((_BK, PADW), lambda i: (i, 0)),
        out_shape=jax.ShapeDtypeStruct((K_ROWS, PADW), jnp.float32),
    )(recip, weight)


_NC = 2
_NS = 16
_NW = _NC * _NS
_B_TOTAL = 4096 * 50
_PER_W = _B_TOTAL // _NW
_RCH = 128
_NG = 5
_R = _RCH * _NG
_CH = _PER_W // _R
_IROWS_W = _PER_W // _RCH


def _gather_body(wpad_hbm, idx_hbm, out_hbm, idx_v, rows_v, sem):
    c = lax.axis_index("c")
    s = lax.axis_index("s")
    wid = s * _NC + c
    rbase = wid * _PER_W

    pltpu.sync_copy(idx_hbm.at[wid], idx_v)

    def chunk(i, carry):
        roff = rbase + i * _R
        cps = [
            pltpu.async_copy(
                wpad_hbm.at[idx_v.at[i * _NG + j]],
                rows_v.at[pl.ds(j * _RCH, _RCH)],
                sem)
            for j in range(_NG)
        ]
        for cp in cps:
            cp.wait()
        pltpu.sync_copy(rows_v, out_hbm.at[pl.ds(roff, _R)])
        return carry

    lax.fori_loop(0, _CH, chunk, 0)


def _gather(wpad, idx2d):
    mesh = plsc.VectorSubcoreMesh(
        core_axis_name="c", subcore_axis_name="s",
        num_cores=_NC, num_subcores=_NS)
    fn = pl.kernel(
        _gather_body,
        out_type=jax.ShapeDtypeStruct((_B_TOTAL, PADW), jnp.float32),
        mesh=mesh,
        scratch_types=[
            pltpu.VMEM((_IROWS_W, _RCH), jnp.int32),
            pltpu.VMEM((_R, PADW), jnp.float32),
            pltpu.SemaphoreType.DMA,
        ],
    )
    return fn(wpad, idx2d)


def kernel(weight, u, k):
    ki = k.astype(jnp.int32)
    recip = _recip_sigma(weight, u)
    wscaled = _pad_scale(weight, recip)
    idx2d = ki.reshape(_NW, _IROWS_W, _RCH)
    g = _gather(wscaled, idx2d)
    return g[:, :NZ].reshape(_NQ, _SEQ, NZ)
